# baseline (speedup 1.0000x reference)
import functools

import jax
import jax.numpy as jnp
from jax import lax
from jax.experimental import pallas as pl
from jax.experimental.pallas import tpu as pltpu
from jax.experimental.pallas import tpu_sc as plsc

NC = 2
NS = 16
NW = NC * NS

BATCH = 1024
SEQ = 200
D = 64
DP = 2 * D
MAX_POS = 201
VOCAB = 1000000
N = BATCH * SEQ
PER_W = N // NW
G = 128
NG = PER_W // G
NBUF = 5
NOB = 2
LANES = 16
QUARTERS = D // LANES

_MESH = plsc.VectorSubcoreMesh(
    core_axis_name="c", subcore_axis_name="s", num_cores=NC, num_subcores=NS
)

TBK = 32768


def _transpose_block(wt_ref, out_ref):
    out_ref[:, pl.ds(0, D)] = wt_ref[...].T


def _transpose_tc(wt):
    v = wt.shape[1]
    grid = (v + TBK - 1) // TBK
    return pl.pallas_call(
        _transpose_block,
        grid=(grid,),
        in_specs=[pl.BlockSpec((D, TBK), lambda i: (0, i))],
        out_specs=pl.BlockSpec((TBK, DP), lambda i: (i, 0)),
        out_shape=jax.ShapeDtypeStruct((v, DP), jnp.float32),
    )(wt)


@functools.partial(
    pl.kernel,
    out_type=jax.ShapeDtypeStruct((N, DP), jnp.float32),
    mesh=_MESH,
    compiler_params=pltpu.CompilerParams(use_tc_tiling_on_sc=False),
    scratch_types=[
        pltpu.VMEM((PER_W,), jnp.int32),
        pltpu.VMEM((PER_W,), jnp.int32),
        pltpu.VMEM((NBUF, G, D), jnp.float32),
        pltpu.VMEM((NBUF, G, D), jnp.float32),
        pltpu.VMEM((NOB, G, D), jnp.float32),
        pltpu.VMEM_SHARED((MAX_POS, D), jnp.float32),
        pltpu.SemaphoreType.DMA((NBUF,)),
        pltpu.SemaphoreType.DMA((NBUF,)),
        pltpu.SemaphoreType.DMA((NOB,)),
    ],
)
def _embed_sc(iid_hbm, pid_hbm, word_hbm, pos_hbm, out_hbm,
              idx_v, pidx_v, wr_v, pr_v, ob_v, pos_sh, sem_w, sem_p, sem_o):
    sid = lax.axis_index("s")
    wid = sid * NC + lax.axis_index("c")
    base = wid * PER_W

    @pl.when(sid == 0)
    def _():
        pltpu.sync_copy(pos_hbm, pos_sh)

    pltpu.sync_copy(iid_hbm.at[pl.ds(base, PER_W)], idx_v)
    pltpu.sync_copy(pid_hbm.at[pl.ds(base, PER_W)], pidx_v)

    def dbl_body(i, c):
        sl = pl.ds(i * LANES, LANES)
        idx_v[sl] = idx_v[sl] * 2
        return c

    lax.fori_loop(0, PER_W // LANES, dbl_body, 0, unroll=4)
    plsc.subcore_barrier()

    def fire_gathers(g, b):
        isl = pl.ds(g * G, G)
        pltpu.async_copy(word_hbm.at[idx_v.at[isl]], wr_v.at[b], sem_w.at[b])
        pltpu.async_copy(pos_sh.at[pidx_v.at[isl]], pr_v.at[b], sem_p.at[b])

    def wait_64(b, sem):
        pltpu.make_async_copy(out_hbm.at[pl.ds(0, G)], pr_v.at[b], sem.at[b]).wait()

    for b in range(NBUF - 1):
        fire_gathers(b, b)

    def outer(j, carry):
        for b in range(NBUF):
            g = j * NBUF + b
            ga = g + NBUF - 1
            sa = (b + NBUF - 1) % NBUF
            ob = g % NOB

            @pl.when(ga < NG)
            def _():
                fire_gathers(ga, sa)

            wait_64(b, sem_w)
            wait_64(b, sem_p)

            @pl.when(g >= NOB)
            def _():
                wait_64(ob, sem_o)

            def add_body(r, c):
                for q in range(QUARTERS):
                    sl = pl.ds(q * LANES, LANES)
                    ob_v[ob, r, sl] = wr_v[b, r, sl] + pr_v[b, r, sl]
                return c

            lax.fori_loop(0, G, add_body, 0, unroll=4)
            pltpu.async_copy(
                ob_v.at[ob],
                out_hbm.at[pl.ds(base + g * G, G), pl.ds(0, D)],
                sem_o.at[ob],
            )
        return carry

    lax.fori_loop(0, NG // NBUF, outer, 0)

    for b in range(NOB):
        wait_64(b, sem_o)


def kernel(input_ids, position_ids, word_embeddings, position_embeddings):
    iid = input_ids.reshape(N)
    pid = position_ids.reshape(N)
    word_rm = _transpose_tc(word_embeddings.T).reshape(2 * VOCAB, D)
    out = _embed_sc(iid, pid, word_rm, position_embeddings)
    return out[:, :D].reshape(BATCH, SEQ, D)


# speedup vs baseline: 1.1288x; 1.1288x over previous
import functools

import jax
import jax.numpy as jnp
from jax import lax
from jax.experimental import pallas as pl
from jax.experimental.pallas import tpu as pltpu
from jax.experimental.pallas import tpu_sc as plsc

NC = 2
NS = 16
NW = NC * NS

BATCH = 1024
SEQ = 200
D = 64
DP = 2 * D
MAX_POS = 201
VOCAB = 1000000
N = BATCH * SEQ
PER_W = N // NW
G = 160
NG = PER_W // G
NBUF = 5
LANES = 16
QUARTERS = D // LANES

_MESH = plsc.VectorSubcoreMesh(
    core_axis_name="c", subcore_axis_name="s", num_cores=NC, num_subcores=NS
)

TBK = 32768


def _transpose_block(wt_ref, out_ref):
    out_ref[:, pl.ds(0, D)] = wt_ref[...].T


def _transpose_tc(wt):
    v = wt.shape[1]
    grid = (v + TBK - 1) // TBK
    return pl.pallas_call(
        _transpose_block,
        grid=(grid,),
        in_specs=[pl.BlockSpec((D, TBK), lambda i: (0, i))],
        out_specs=pl.BlockSpec((TBK, DP), lambda i: (i, 0)),
        out_shape=jax.ShapeDtypeStruct((v, DP), jnp.float32),
    )(wt)


@functools.partial(
    pl.kernel,
    out_type=jax.ShapeDtypeStruct((N, DP), jnp.float32),
    mesh=_MESH,
    compiler_params=pltpu.CompilerParams(use_tc_tiling_on_sc=False),
    scratch_types=[
        pltpu.VMEM((PER_W,), jnp.int32),
        pltpu.VMEM((PER_W,), jnp.int32),
        pltpu.VMEM((NBUF, G, D), jnp.float32),
        pltpu.VMEM((NBUF, G, D), jnp.float32),
        pltpu.VMEM_SHARED((MAX_POS, D), jnp.float32),
        pltpu.SemaphoreType.DMA((NBUF,)),
        pltpu.SemaphoreType.DMA((NBUF,)),
        pltpu.SemaphoreType.DMA((NBUF,)),
    ],
)
def _embed_sc(iid_hbm, pid_hbm, word_hbm, pos_hbm, out_hbm,
              idx_v, pidx_v, wr_v, pr_v, pos_sh, sem_w, sem_p, sem_o):
    sid = lax.axis_index("s")
    wid = sid * NC + lax.axis_index("c")
    base = wid * PER_W

    @pl.when(sid == 0)
    def _():
        pltpu.sync_copy(pos_hbm, pos_sh)

    pltpu.sync_copy(iid_hbm.at[pl.ds(base, PER_W)], idx_v)
    pltpu.sync_copy(pid_hbm.at[pl.ds(base, PER_W)], pidx_v)

    def dbl_body(i, c):
        sl = pl.ds(i * LANES, LANES)
        idx_v[sl] = idx_v[sl] * 2
        return c

    lax.fori_loop(0, PER_W // LANES, dbl_body, 0, unroll=4)
    plsc.subcore_barrier()

    def fire_gathers(g, b):
        isl = pl.ds(g * G, G)
        pltpu.async_copy(word_hbm.at[idx_v.at[isl]], wr_v.at[b], sem_w.at[b])
        pltpu.async_copy(pos_sh.at[pidx_v.at[isl]], pr_v.at[b], sem_p.at[b])

    def wait_64(b, sem):
        pltpu.make_async_copy(out_hbm.at[pl.ds(0, G)], pr_v.at[b], sem.at[b]).wait()

    for b in range(NBUF - 1):
        fire_gathers(b, b)

    def outer(j, carry):
        for b in range(NBUF):
            g = j * NBUF + b
            ga = g + NBUF - 1
            sa = (b + NBUF - 1) % NBUF

            @pl.when(jnp.logical_and(ga >= NBUF, ga < NG))
            def _():
                wait_64(sa, sem_o)

            @pl.when(ga < NG)
            def _():
                fire_gathers(ga, sa)

            wait_64(b, sem_w)
            wait_64(b, sem_p)

            def add_body(r, c):
                for q in range(QUARTERS):
                    sl = pl.ds(q * LANES, LANES)
                    wr_v[b, r, sl] = wr_v[b, r, sl] + pr_v[b, r, sl]
                return c

            lax.fori_loop(0, G, add_body, 0, unroll=4)
            pltpu.async_copy(
                wr_v.at[b],
                out_hbm.at[pl.ds(base + g * G, G), pl.ds(0, D)],
                sem_o.at[b],
            )
        return carry

    lax.fori_loop(0, NG // NBUF, outer, 0)

    for b in range(NBUF):
        wait_64(b, sem_o)


def kernel(input_ids, position_ids, word_embeddings, position_embeddings):
    iid = input_ids.reshape(N)
    pid = position_ids.reshape(N)
    word_rm = _transpose_tc(word_embeddings.T).reshape(2 * VOCAB, D)
    out = _embed_sc(iid, pid, word_rm, position_embeddings)
    return out[:, :D].reshape(BATCH, SEQ, D)


# speedup vs baseline: 1.1300x; 1.0010x over previous
import functools

import jax
import jax.numpy as jnp
from jax import lax
from jax.experimental import pallas as pl
from jax.experimental.pallas import tpu as pltpu
from jax.experimental.pallas import tpu_sc as plsc

NC = 2
NS = 16
NW = NC * NS

BATCH = 1024
SEQ = 200
D = 64
DP = 2 * D
MAX_POS = 201
VOCAB = 1000000
N = BATCH * SEQ
PER_W = N // NW
G = 160
NG = PER_W // G
NBUF = 5
LANES = 16
QUARTERS = D // LANES

_MESH = plsc.VectorSubcoreMesh(
    core_axis_name="c", subcore_axis_name="s", num_cores=NC, num_subcores=NS
)

TBK = 32768


def _transpose_block(wt_ref, out_ref):
    out_ref[:, pl.ds(0, D)] = wt_ref[...].T


def _transpose_tc(wt):
    v = wt.shape[1]
    grid = (v + TBK - 1) // TBK
    return pl.pallas_call(
        _transpose_block,
        grid=(grid,),
        in_specs=[pl.BlockSpec((D, TBK), lambda i: (0, i))],
        out_specs=pl.BlockSpec((TBK, DP), lambda i: (i, 0)),
        out_shape=jax.ShapeDtypeStruct((v, DP), jnp.float32),
    )(wt)


@functools.partial(
    pl.kernel,
    out_type=jax.ShapeDtypeStruct((N, DP), jnp.float32),
    mesh=_MESH,
    compiler_params=pltpu.CompilerParams(use_tc_tiling_on_sc=False),
    scratch_types=[
        pltpu.VMEM((PER_W,), jnp.int32),
        pltpu.VMEM((PER_W,), jnp.int32),
        pltpu.VMEM((NBUF, G, D), jnp.float32),
        pltpu.VMEM((NBUF, G, D), jnp.float32),
        pltpu.VMEM_SHARED((MAX_POS, D), jnp.float32),
        pltpu.SemaphoreType.DMA((NBUF,)),
        pltpu.SemaphoreType.DMA((NBUF,)),
        pltpu.SemaphoreType.DMA((NBUF,)),
    ],
)
def _embed_sc(iid_hbm, pid_hbm, word_hbm, pos_hbm, out_hbm,
              idx_v, pidx_v, wr_v, pr_v, pos_sh, sem_w, sem_p, sem_o):
    sid = lax.axis_index("s")
    wid = sid * NC + lax.axis_index("c")
    base = wid * PER_W

    @pl.when(sid == 0)
    def _():
        pltpu.sync_copy(pos_hbm, pos_sh)

    pltpu.sync_copy(iid_hbm.at[pl.ds(base, PER_W)], idx_v)
    pltpu.sync_copy(pid_hbm.at[pl.ds(base, PER_W)], pidx_v)
    plsc.subcore_barrier()

    def fire_gathers(g, b):
        isl = pl.ds(g * G, G)
        pltpu.async_copy(word_hbm.at[idx_v.at[isl]], wr_v.at[b], sem_w.at[b])
        pltpu.async_copy(pos_sh.at[pidx_v.at[isl]], pr_v.at[b], sem_p.at[b])

    def wait_64(b, sem):
        pltpu.make_async_copy(out_hbm.at[pl.ds(0, G)], pr_v.at[b], sem.at[b]).wait()

    for b in range(NBUF - 1):
        fire_gathers(b, b)

    def outer(j, carry):
        for b in range(NBUF):
            g = j * NBUF + b
            ga = g + NBUF - 1
            sa = (b + NBUF - 1) % NBUF

            @pl.when(jnp.logical_and(ga >= NBUF, ga < NG))
            def _():
                wait_64(sa, sem_o)

            @pl.when(ga < NG)
            def _():
                fire_gathers(ga, sa)

            wait_64(b, sem_w)
            wait_64(b, sem_p)

            def add_body(r, c):
                for q in range(QUARTERS):
                    sl = pl.ds(q * LANES, LANES)
                    wr_v[b, r, sl] = wr_v[b, r, sl] + pr_v[b, r, sl]
                return c

            lax.fori_loop(0, G, add_body, 0, unroll=4)
            pltpu.async_copy(
                wr_v.at[b],
                out_hbm.at[pl.ds(base + g * G, G), pl.ds(0, D)],
                sem_o.at[b],
            )
        return carry

    lax.fori_loop(0, NG // NBUF, outer, 0)

    for b in range(NBUF):
        wait_64(b, sem_o)


def kernel(input_ids, position_ids, word_embeddings, position_embeddings):
    iid = input_ids.reshape(N) * 2
    pid = position_ids.reshape(N)
    word_rm = _transpose_tc(word_embeddings.T).reshape(2 * VOCAB, D)
    out = _embed_sc(iid, pid, word_rm, position_embeddings)
    return out[:, :D].reshape(BATCH, SEQ, D)
